# Initial kernel scaffold; baseline (speedup 1.0000x reference)
import jax
import jax.numpy as jnp
from jax.experimental import pallas as pl


def kernel(x):
    raise NotImplementedError("write your pallas kernel here")


# speedup vs baseline: 114.9629x; 114.9629x over previous
import functools

import jax
import jax.numpy as jnp
from jax import lax
from jax.experimental import pallas as pl
from jax.experimental.pallas import tpu as pltpu
from jax.experimental.pallas import tpu_sc as plsc

B = 1024
P = B * (B - 1) // 2
NW = 32
CHUNK = P // NW
VECS = CHUNK // 16
TWO_B_M1 = 2 * B - 1
DISC0 = TWO_B_M1 * TWO_B_M1


def _row_offset(i):
    return (i * (TWO_B_M1 - i)) >> 1


def _pairs_body(x_hbm, x1_hbm, x2_hbm, x_v, o1_v, o2_v):
    wid = lax.axis_index("s") * 2 + lax.axis_index("c")
    base = wid * CHUNK
    pltpu.sync_copy(x_hbm, x_v)
    lane = lax.iota(jnp.int32, 16)

    def body(t, carry):
        k = (base + t * 16) + lane
        disc = jnp.int32(DISC0) - 8 * k
        df = disc.astype(jnp.float32)
        bits = plsc.bitcast(df, jnp.int32)
        r = plsc.bitcast(jnp.int32(0x5F3759DF) - (bits >> 1), jnp.float32)
        hd = jnp.float32(0.5) * df
        r = r * (jnp.float32(1.5) - hd * r * r)
        r = r * (jnp.float32(1.5) - hd * r * r)
        s = df * r
        i0 = ((jnp.float32(TWO_B_M1) - s) * jnp.float32(0.5)).astype(jnp.int32)
        i0 = jnp.where(_row_offset(i0 + 1) <= k, i0 + 1, i0)
        i0 = jnp.where(_row_offset(i0 + 1) <= k, i0 + 1, i0)
        i0 = jnp.where(_row_offset(i0) > k, i0 - 1, i0)
        i0 = jnp.where(_row_offset(i0) > k, i0 - 1, i0)
        j = (k - _row_offset(i0)) + (i0 + 1)
        o1_v[pl.ds(t * 16, 16)] = plsc.load_gather(x_v, [i0])
        o2_v[pl.ds(t * 16, 16)] = plsc.load_gather(x_v, [j])
        return carry

    lax.fori_loop(0, VECS, body, 0, unroll=4)
    pltpu.sync_copy(o1_v, x1_hbm.at[pl.ds(base, CHUNK)])
    pltpu.sync_copy(o2_v, x2_hbm.at[pl.ds(base, CHUNK)])


@functools.cache
def _build():
    return functools.partial(
        pl.kernel,
        out_type=(
            jax.ShapeDtypeStruct((P,), jnp.float32),
            jax.ShapeDtypeStruct((P,), jnp.float32),
        ),
        mesh=plsc.VectorSubcoreMesh(
            core_axis_name="c", subcore_axis_name="s", num_cores=2, num_subcores=16
        ),
        scratch_types=[
            pltpu.VMEM((B,), jnp.float32),
            pltpu.VMEM((CHUNK,), jnp.float32),
            pltpu.VMEM((CHUNK,), jnp.float32),
        ],
        compiler_params=pltpu.CompilerParams(needs_layout_passes=False),
    )(_pairs_body)


def kernel(x):
    return _build()(x)
